# baseline (speedup 1.0000x reference)
import functools

import jax
import jax.numpy as jnp
from jax import lax
from jax.experimental import pallas as pl
from jax.experimental.pallas import tpu as pltpu
from jax.experimental.pallas import tpu_sc as plsc

_NC = 512
_T = 256
_TK = 256
_DS = 128


def _ln(x, g, b):
    m = jnp.mean(x, axis=-1, keepdims=True)
    v = jnp.mean((x - m) * (x - m), axis=-1, keepdims=True)
    return (x - m) * jax.lax.rsqrt(v + 1e-5) * g + b


def _sc_body(pos_hbm, idxl_hbm, idxg_hbm, zeros_hbm, sums_hbm, cogg_hbm,
             pos_v, idxl_v, idxg_v, rows_v, acc32_v, sem, acc_sh):
    c = lax.axis_index("c")
    s = lax.axis_index("s")

    @pl.when(s == 0)
    def _():
        pltpu.sync_copy(zeros_hbm, acc_sh)
    plsc.subcore_barrier()

    base = c * 32 + s * 2
    pltpu.sync_copy(pos_hbm.at[pl.ds(base * 128, 256)], pos_v)
    pltpu.sync_copy(idxl_hbm.at[pl.ds(base, 2)], idxl_v)
    pltpu.sync_copy(idxg_hbm.at[pl.ds(base, 2)], idxg_v)
    for j in range(2):
        pltpu.sync_copy(pos_v.at[pl.ds(j * 128, 128)],
                        acc_sh.at[idxl_v.at[j]], add=True)
    plsc.subcore_barrier()

    pltpu.sync_copy(acc_sh.at[pl.ds(s * 32, 32)], acc32_v)
    pltpu.sync_copy(acc32_v, sums_hbm.at[pl.ds(c * _NC + s * 32, 32)])
    plsc.subcore_barrier()
    for j in range(2):
        pltpu.async_copy(sums_hbm.at[idxg_v.at[j]], rows_v, sem).wait()
        pltpu.sync_copy(rows_v, cogg_hbm.at[pl.ds((base + j) * 128, 128)])


def _sc_segment(pos16, idx_local, idx_global, zeros):
    BN = pos16.shape[0]
    mesh = plsc.VectorSubcoreMesh(core_axis_name="c", subcore_axis_name="s")
    f32 = jnp.float32
    run = pl.kernel(
        _sc_body,
        mesh=mesh,
        out_type=[jax.ShapeDtypeStruct((2 * _NC, _DS), f32),
                  jax.ShapeDtypeStruct((BN, _DS), f32)],
        scratch_types=[
            pltpu.VMEM((256, _DS), f32),
            pltpu.VMEM((2, 128), jnp.int32),
            pltpu.VMEM((2, 128), jnp.int32),
            pltpu.VMEM((128, _DS), f32),
            pltpu.VMEM((32, _DS), f32),
            pltpu.SemaphoreType.DMA,
            pltpu.VMEM_SHARED((_NC, _DS), f32),
        ],
    )
    _, cogg = run(pos16, idx_local, idx_global, zeros)
    return cogg


def _tc_body(pos_ref, cogg_ref, feat_ref, cidr_ref, jlo_ref, jhi_ref,
             w1a_ref, w1an_ref, b1a_ref, g1a_ref, be1a_ref,
             w1br_ref, w1bf_ref, b1b_ref, g1b_ref, be1b_ref,
             w2a_ref, b2a_ref, g2a_ref, be2a_ref,
             w2br_ref, w2bf_ref, b2b_ref, g2b_ref, be2b_ref,
             wq_ref, wk_ref, wv_ref, wo_ref, bo_ref, gl_ref, bl_ref,
             out_ref,
             hpos_ref, q_ref, k_ref, v_ref, acc_ref, m_ref, l_ref):
    N = pos_ref.shape[1]
    NT = N // _T
    b = pl.program_id(0)
    f32 = jnp.float32

    def dot(a, bb, dims):
        return lax.dot_general(a, bb, (dims, ((), ())),
                               preferred_element_type=f32)

    def p1_tile(t):
        sl = pl.ds(t * _T, _T)
        pt = pos_ref[0, sl, :]
        cg = cogg_ref[0, sl, :]
        cnt = jnp.maximum(cg[:, 3:4], 1.0)
        lp = pt - cg / cnt
        n = jnp.sqrt(jnp.sum(lp * lp, axis=1, keepdims=True))
        x1 = dot(lp, w1a_ref[...], ((1,), (0,))) + n * w1an_ref[...] + b1a_ref[...]
        r = jax.nn.relu(_ln(x1, g1a_ref[...], be1a_ref[...]))
        ft = feat_ref[0, sl, :]
        h1 = (dot(r, w1br_ref[...], ((1,), (0,)))
              + dot(ft, w1bf_ref[...], ((1,), (0,))) + b1b_ref[...])
        hpos = jax.nn.relu(_ln(h1, g1b_ref[...], be1b_ref[...]))
        x2 = dot(lp, w2a_ref[...], ((1,), (0,))) + b2a_ref[...]
        rh = jax.nn.relu(_ln(x2, g2a_ref[...], be2a_ref[...]))
        h2 = (dot(rh, w2br_ref[...], ((1,), (0,)))
              + dot(ft, w2bf_ref[...], ((1,), (0,))) + b2b_ref[...])
        hgeo = jax.nn.relu(_ln(h2, g2b_ref[...], be2b_ref[...]))
        hpos_ref[sl, :] = hpos
        q_ref[sl, :] = dot(hgeo, wq_ref[...], ((1,), (0,))) * (1.0 / 16.0)
        k_ref[sl, :] = dot(hgeo, wk_ref[...], ((1,), (0,)))
        v_ref[sl, :] = dot(hpos, wv_ref[...], ((1,), (0,)))

    def p1(t, _):
        p1_tile(2 * t)
        p1_tile(2 * t + 1)
        return 0

    lax.fori_loop(0, NT // 2, p1, 0)

    def p2(t, _):
        sl = pl.ds(t * _T, _T)
        acc_ref[...] = jnp.zeros((_T, 256), f32)
        m_ref[...] = jnp.full((_T, 1), -1e20, f32)
        l_ref[...] = jnp.zeros((_T, 1), f32)
        cq = pos_ref[0, sl, 4:5]

        def jb(j, _):
            off = pl.multiple_of(j * _TK, _TK)
            ksl = pl.ds(off, _TK)
            s = dot(q_ref[sl, :], k_ref[ksl, :], ((1,), (1,)))
            ck = cidr_ref[0, :, ksl]
            s = jnp.where(cq == ck, s, -1e30)
            mprev = m_ref[...]
            mnew = jnp.maximum(mprev, jnp.max(s, axis=1, keepdims=True))
            p = jnp.exp(s - mnew)
            alpha = jnp.exp(mprev - mnew)
            l_ref[...] = l_ref[...] * alpha + jnp.sum(p, axis=1, keepdims=True)
            acc_ref[...] = acc_ref[...] * alpha + dot(p, v_ref[ksl, :], ((1,), (0,)))
            m_ref[...] = mnew
            return 0

        lax.fori_loop(jlo_ref[b, t], jhi_ref[b, t] + 1, jb, 0)
        o = acc_ref[...] / l_ref[...]
        attn = dot(o, wo_ref[...], ((1,), (0,))) + bo_ref[...]
        y = hpos_ref[sl, :] + attn
        out_ref[0, sl, :] = _ln(y, gl_ref[...], bl_ref[...])
        return 0

    lax.fori_loop(0, NT, p2, 0)


def kernel(pos, feat, cluster_ids, W1a, b1a, g1a, be1a, W1b, b1b, g1b, be1b,
           W2a, b2a, g2a, be2a, W2b, b2b, g2b, be2b, Wq, Wk, Wv, Wo, bo,
           g_ln1, b_ln1):
    B, N, _ = pos.shape
    D_PE = W1a.shape[1]
    D_FEAT = feat.shape[2]
    D_EMB = W1b.shape[1]
    NT = N // _T
    f32 = jnp.float32

    cid = cluster_ids.astype(jnp.int32)
    pos16 = jnp.zeros((B * N, _DS), f32).at[:, :3].set(pos.reshape(B * N, 3))
    pos16 = pos16.at[:, 3].set(1.0)
    pos16 = pos16.at[:, 4].set(cid.reshape(B * N).astype(f32))
    idx_local = cid.reshape(B * N // 128, 128)
    idx_global = (cid + jnp.arange(B, dtype=jnp.int32)[:, None] * _NC
                  ).reshape(B * N // 128, 128)
    zeros = jnp.zeros((_NC, _DS), f32)

    cogg = _sc_segment(pos16, idx_local, idx_global, zeros).reshape(B, N, _DS)

    cid_row = cid.reshape(B, 1, N).astype(f32)

    qs = jnp.arange(NT) * _T
    c_first = cid[:, qs]
    c_last = cid[:, qs + _T - 1]
    lo = jax.vmap(lambda a, v: jnp.searchsorted(a, v, side='left'))(cid, c_first)
    hi = jax.vmap(lambda a, v: jnp.searchsorted(a, v, side='right'))(cid, c_last)
    jlo = (lo // _TK).astype(jnp.int32)
    jhi = ((hi - 1) // _TK).astype(jnp.int32)

    W1a_pad = jnp.zeros((_DS, D_PE), f32).at[:3].set(W1a[:3])
    w1a_n = W1a[3:4]
    W2a_pad = jnp.zeros((_DS, D_PE), f32).at[:3].set(W2a[3:6])
    row = lambda x: x.reshape(1, -1)

    grid = (B,)
    full = lambda shp: pl.BlockSpec(shp, lambda b: (0,) * len(shp))
    batched = lambda shp: pl.BlockSpec((1,) + shp, lambda b: (b, 0, 0))

    out = pl.pallas_call(
        _tc_body,
        grid=grid,
        in_specs=[
            batched((N, _DS)),
            batched((N, _DS)),
            batched((N, D_FEAT)),
            batched((1, N)),
            pl.BlockSpec(memory_space=pltpu.SMEM),
            pl.BlockSpec(memory_space=pltpu.SMEM),
            full((_DS, D_PE)), full((1, D_PE)), full((1, D_PE)),
            full((1, D_PE)), full((1, D_PE)),
            full((D_PE, D_EMB)), full((D_FEAT, D_EMB)), full((1, D_EMB)),
            full((1, D_EMB)), full((1, D_EMB)),
            full((_DS, D_PE)), full((1, D_PE)), full((1, D_PE)), full((1, D_PE)),
            full((D_PE, D_EMB)), full((D_FEAT, D_EMB)), full((1, D_EMB)),
            full((1, D_EMB)), full((1, D_EMB)),
            full((D_EMB, D_EMB)), full((D_EMB, D_EMB)), full((D_EMB, D_EMB)),
            full((D_EMB, D_EMB)), full((1, D_EMB)), full((1, D_EMB)),
            full((1, D_EMB)),
        ],
        out_specs=batched((N, D_EMB)),
        out_shape=jax.ShapeDtypeStruct((B, N, D_EMB), f32),
        scratch_shapes=[
            pltpu.VMEM((N, D_EMB), f32),
            pltpu.VMEM((N, D_EMB), f32),
            pltpu.VMEM((N, D_EMB), f32),
            pltpu.VMEM((N, D_EMB), f32),
            pltpu.VMEM((_T, D_EMB), f32),
            pltpu.VMEM((_T, 1), f32),
            pltpu.VMEM((_T, 1), f32),
        ],
    )(pos16.reshape(B, N, _DS), cogg, feat, cid_row, jlo, jhi,
      W1a_pad, w1a_n, row(b1a), row(g1a), row(be1a),
      W1b[:D_PE], W1b[D_PE:], row(b1b), row(g1b), row(be1b),
      W2a_pad, row(b2a), row(g2a), row(be2a),
      W2b[:D_PE], W2b[D_PE:], row(b2b), row(g2b), row(be2b),
      Wq, Wk, Wv, Wo, row(bo), row(g_ln1), row(b_ln1))
    return out


# speedup vs baseline: 1.0149x; 1.0149x over previous
import functools

import jax
import jax.numpy as jnp
from jax import lax
from jax.experimental import pallas as pl
from jax.experimental.pallas import tpu as pltpu
from jax.experimental.pallas import tpu_sc as plsc

_NC = 512
_T = 256
_TK = 128
_DS = 128


def _ln(x, g, b):
    m = jnp.mean(x, axis=-1, keepdims=True)
    v = jnp.mean((x - m) * (x - m), axis=-1, keepdims=True)
    return (x - m) * jax.lax.rsqrt(v + 1e-5) * g + b


def _sc_body(pos_hbm, idxl_hbm, idxg_hbm, zeros_hbm, sums_hbm, cogg_hbm,
             pos_v, idxl_v, idxg_v, rows_v, acc32_v, sem, acc_sh):
    c = lax.axis_index("c")
    s = lax.axis_index("s")

    @pl.when(s == 0)
    def _():
        pltpu.sync_copy(zeros_hbm, acc_sh)
    plsc.subcore_barrier()

    base = c * 32 + s * 2
    pltpu.sync_copy(pos_hbm.at[pl.ds(base * 128, 256)], pos_v)
    pltpu.sync_copy(idxl_hbm.at[pl.ds(base, 2)], idxl_v)
    pltpu.sync_copy(idxg_hbm.at[pl.ds(base, 2)], idxg_v)
    for j in range(2):
        pltpu.sync_copy(pos_v.at[pl.ds(j * 128, 128)],
                        acc_sh.at[idxl_v.at[j]], add=True)
    plsc.subcore_barrier()

    pltpu.sync_copy(acc_sh.at[pl.ds(s * 32, 32)], acc32_v)
    pltpu.sync_copy(acc32_v, sums_hbm.at[pl.ds(c * _NC + s * 32, 32)])
    plsc.subcore_barrier()
    for j in range(2):
        pltpu.async_copy(sums_hbm.at[idxg_v.at[j]], rows_v, sem).wait()
        pltpu.sync_copy(rows_v, cogg_hbm.at[pl.ds((base + j) * 128, 128)])


def _sc_segment(pos16, idx_local, idx_global, zeros):
    BN = pos16.shape[0]
    mesh = plsc.VectorSubcoreMesh(core_axis_name="c", subcore_axis_name="s")
    f32 = jnp.float32
    run = pl.kernel(
        _sc_body,
        mesh=mesh,
        out_type=[jax.ShapeDtypeStruct((2 * _NC, _DS), f32),
                  jax.ShapeDtypeStruct((BN, _DS), f32)],
        scratch_types=[
            pltpu.VMEM((256, _DS), f32),
            pltpu.VMEM((2, 128), jnp.int32),
            pltpu.VMEM((2, 128), jnp.int32),
            pltpu.VMEM((128, _DS), f32),
            pltpu.VMEM((32, _DS), f32),
            pltpu.SemaphoreType.DMA,
            pltpu.VMEM_SHARED((_NC, _DS), f32),
        ],
    )
    _, cogg = run(pos16, idx_local, idx_global, zeros)
    return cogg


def _tc_body(pos_ref, cogg_ref, feat_ref, cidr_ref, cidc_ref, jlo_ref, jhi_ref,
             w1a_ref, w1an_ref, b1a_ref, g1a_ref, be1a_ref,
             w1br_ref, w1bf_ref, b1b_ref, g1b_ref, be1b_ref,
             w2a_ref, b2a_ref, g2a_ref, be2a_ref,
             w2br_ref, w2bf_ref, b2b_ref, g2b_ref, be2b_ref,
             wq_ref, wk_ref, wv_ref, wo_ref, bo_ref, gl_ref, bl_ref,
             out_ref,
             hpos_ref, q_ref, k_ref, v_ref, acc_ref, m_ref, l_ref):
    N = pos_ref.shape[1]
    NT = N // _T
    b = pl.program_id(0)
    f32 = jnp.float32

    def dot(a, bb, dims):
        return lax.dot_general(a, bb, (dims, ((), ())),
                               preferred_element_type=f32)

    def p1_tile(t):
        sl = pl.ds(t * _T, _T)
        pt = pos_ref[0, sl, :]
        cg = cogg_ref[0, sl, :]
        cnt = jnp.maximum(cg[:, 3:4], 1.0)
        lp = pt - cg / cnt
        n = jnp.sqrt(jnp.sum(lp * lp, axis=1, keepdims=True))
        x1 = dot(lp, w1a_ref[...], ((1,), (0,))) + n * w1an_ref[...] + b1a_ref[...]
        r = jax.nn.relu(_ln(x1, g1a_ref[...], be1a_ref[...]))
        ft = feat_ref[0, sl, :]
        h1 = (dot(r, w1br_ref[...], ((1,), (0,)))
              + dot(ft, w1bf_ref[...], ((1,), (0,))) + b1b_ref[...])
        hpos = jax.nn.relu(_ln(h1, g1b_ref[...], be1b_ref[...]))
        x2 = dot(lp, w2a_ref[...], ((1,), (0,))) + b2a_ref[...]
        rh = jax.nn.relu(_ln(x2, g2a_ref[...], be2a_ref[...]))
        h2 = (dot(rh, w2br_ref[...], ((1,), (0,)))
              + dot(ft, w2bf_ref[...], ((1,), (0,))) + b2b_ref[...])
        hgeo = jax.nn.relu(_ln(h2, g2b_ref[...], be2b_ref[...]))
        hpos_ref[sl, :] = hpos
        q_ref[sl, :] = dot(hgeo, wq_ref[...], ((1,), (0,))) * (1.0 / 16.0)
        k_ref[sl, :] = dot(hgeo, wk_ref[...], ((1,), (0,)))
        v_ref[sl, :] = dot(hpos, wv_ref[...], ((1,), (0,)))

    def p1(t, _):
        p1_tile(2 * t)
        p1_tile(2 * t + 1)
        return 0

    lax.fori_loop(0, NT // 2, p1, 0)

    def p2(t, _):
        sl = pl.ds(t * _T, _T)
        acc_ref[...] = jnp.zeros((_T, 256), f32)
        m_ref[...] = jnp.full((_T, 1), -1e20, f32)
        l_ref[...] = jnp.zeros((_T, 1), f32)
        cq = cidc_ref[0, sl, :]

        def jb(j, _):
            off = pl.multiple_of(j * _TK, _TK)
            ksl = pl.ds(off, _TK)
            s = dot(q_ref[sl, :], k_ref[ksl, :], ((1,), (1,)))
            ck = cidr_ref[0, :, ksl]
            s = jnp.where(cq == ck, s, -1e30)
            mprev = m_ref[...]
            mnew = jnp.maximum(mprev, jnp.max(s, axis=1, keepdims=True))
            p = jnp.exp(s - mnew)
            alpha = jnp.exp(mprev - mnew)
            l_ref[...] = l_ref[...] * alpha + jnp.sum(p, axis=1, keepdims=True)
            acc_ref[...] = acc_ref[...] * alpha + dot(p, v_ref[ksl, :], ((1,), (0,)))
            m_ref[...] = mnew
            return 0

        lax.fori_loop(jlo_ref[b, t], jhi_ref[b, t] + 1, jb, 0)
        o = acc_ref[...] / l_ref[...]
        attn = dot(o, wo_ref[...], ((1,), (0,))) + bo_ref[...]
        y = hpos_ref[sl, :] + attn
        out_ref[0, sl, :] = _ln(y, gl_ref[...], bl_ref[...])
        return 0

    lax.fori_loop(0, NT, p2, 0)


def kernel(pos, feat, cluster_ids, W1a, b1a, g1a, be1a, W1b, b1b, g1b, be1b,
           W2a, b2a, g2a, be2a, W2b, b2b, g2b, be2b, Wq, Wk, Wv, Wo, bo,
           g_ln1, b_ln1):
    B, N, _ = pos.shape
    D_PE = W1a.shape[1]
    D_FEAT = feat.shape[2]
    D_EMB = W1b.shape[1]
    NT = N // _T
    f32 = jnp.float32

    pos16 = jnp.zeros((B * N, _DS), f32).at[:, :3].set(pos.reshape(B * N, 3))
    pos16 = pos16.at[:, 3].set(1.0)
    cid = cluster_ids.astype(jnp.int32)
    idx_local = cid.reshape(B * N // 128, 128)
    idx_global = (cid + jnp.arange(B, dtype=jnp.int32)[:, None] * _NC
                  ).reshape(B * N // 128, 128)
    zeros = jnp.zeros((_NC, _DS), f32)

    cogg = _sc_segment(pos16, idx_local, idx_global, zeros).reshape(B, N, _DS)

    cid_row = cid.reshape(B, 1, N)
    cid_col = cid.reshape(B, N, 1)

    qs = jnp.arange(NT) * _T
    c_first = cid[:, qs]
    c_last = cid[:, qs + _T - 1]
    lo = jax.vmap(lambda a, v: jnp.searchsorted(a, v, side='left'))(cid, c_first)
    hi = jax.vmap(lambda a, v: jnp.searchsorted(a, v, side='right'))(cid, c_last)
    jlo = (lo // _TK).astype(jnp.int32)
    jhi = ((hi - 1) // _TK).astype(jnp.int32)

    W1a_pad = jnp.zeros((_DS, D_PE), f32).at[:3].set(W1a[:3])
    w1a_n = W1a[3:4]
    W2a_pad = jnp.zeros((_DS, D_PE), f32).at[:3].set(W2a[3:6])
    row = lambda x: x.reshape(1, -1)

    grid = (B,)
    full = lambda shp: pl.BlockSpec(shp, lambda b: (0,) * len(shp))
    batched = lambda shp: pl.BlockSpec((1,) + shp, lambda b: (b, 0, 0))

    out = pl.pallas_call(
        _tc_body,
        grid=grid,
        in_specs=[
            batched((N, _DS)),
            batched((N, _DS)),
            batched((N, D_FEAT)),
            batched((1, N)),
            batched((N, 1)),
            pl.BlockSpec(memory_space=pltpu.SMEM),
            pl.BlockSpec(memory_space=pltpu.SMEM),
            full((_DS, D_PE)), full((1, D_PE)), full((1, D_PE)),
            full((1, D_PE)), full((1, D_PE)),
            full((D_PE, D_EMB)), full((D_FEAT, D_EMB)), full((1, D_EMB)),
            full((1, D_EMB)), full((1, D_EMB)),
            full((_DS, D_PE)), full((1, D_PE)), full((1, D_PE)), full((1, D_PE)),
            full((D_PE, D_EMB)), full((D_FEAT, D_EMB)), full((1, D_EMB)),
            full((1, D_EMB)), full((1, D_EMB)),
            full((D_EMB, D_EMB)), full((D_EMB, D_EMB)), full((D_EMB, D_EMB)),
            full((D_EMB, D_EMB)), full((1, D_EMB)), full((1, D_EMB)),
            full((1, D_EMB)),
        ],
        out_specs=batched((N, D_EMB)),
        out_shape=jax.ShapeDtypeStruct((B, N, D_EMB), f32),
        scratch_shapes=[
            pltpu.VMEM((N, D_EMB), f32),
            pltpu.VMEM((N, D_EMB), f32),
            pltpu.VMEM((N, D_EMB), f32),
            pltpu.VMEM((N, D_EMB), f32),
            pltpu.VMEM((_T, D_EMB), f32),
            pltpu.VMEM((_T, 1), f32),
            pltpu.VMEM((_T, 1), f32),
        ],
    )(pos16.reshape(B, N, _DS), cogg, feat, cid_row, cid_col, jlo, jhi,
      W1a_pad, w1a_n, row(b1a), row(g1a), row(be1a),
      W1b[:D_PE], W1b[D_PE:], row(b1b), row(g1b), row(be1b),
      W2a_pad, row(b2a), row(g2a), row(be2a),
      W2b[:D_PE], W2b[D_PE:], row(b2b), row(g2b), row(be2b),
      Wq, Wk, Wv, Wo, row(bo), row(g_ln1), row(b_ln1))
    return out


# speedup vs baseline: 1.0607x; 1.0451x over previous
import functools

import jax
import jax.numpy as jnp
from jax import lax
from jax.experimental import pallas as pl
from jax.experimental.pallas import tpu as pltpu
from jax.experimental.pallas import tpu_sc as plsc

_NC = 512
_T = 256
_TK = 256
_DS = 128


def _ln(x, g, b):
    m = jnp.mean(x, axis=-1, keepdims=True)
    v = jnp.mean((x - m) * (x - m), axis=-1, keepdims=True)
    return (x - m) * jax.lax.rsqrt(v + 1e-5) * g + b


def _sc_body(pos_hbm, idxl_hbm, idxg_hbm, zeros_hbm, sums_hbm, cogg_hbm,
             pos_v, idxl_v, idxg_v, rows_v, acc32_v, sem, acc_sh):
    c = lax.axis_index("c")
    s = lax.axis_index("s")

    @pl.when(s == 0)
    def _():
        pltpu.sync_copy(zeros_hbm, acc_sh)
    plsc.subcore_barrier()

    base = c * 32 + s * 2
    pltpu.sync_copy(pos_hbm.at[pl.ds(base * 128, 256)], pos_v)
    pltpu.sync_copy(idxl_hbm.at[pl.ds(base, 2)], idxl_v)
    pltpu.sync_copy(idxg_hbm.at[pl.ds(base, 2)], idxg_v)
    for j in range(2):
        pltpu.sync_copy(pos_v.at[pl.ds(j * 128, 128)],
                        acc_sh.at[idxl_v.at[j]], add=True)
    plsc.subcore_barrier()

    pltpu.sync_copy(acc_sh.at[pl.ds(s * 32, 32)], acc32_v)
    pltpu.sync_copy(acc32_v, sums_hbm.at[pl.ds(c * _NC + s * 32, 32)])
    plsc.subcore_barrier()
    for j in range(2):
        pltpu.async_copy(sums_hbm.at[idxg_v.at[j]], rows_v, sem).wait()
        pltpu.sync_copy(rows_v, cogg_hbm.at[pl.ds((base + j) * 128, 128)])


def _sc_segment(pos16, idx_local, idx_global, zeros):
    BN = pos16.shape[0]
    mesh = plsc.VectorSubcoreMesh(core_axis_name="c", subcore_axis_name="s")
    f32 = jnp.float32
    run = pl.kernel(
        _sc_body,
        mesh=mesh,
        out_type=[jax.ShapeDtypeStruct((2 * _NC, _DS), f32),
                  jax.ShapeDtypeStruct((BN, _DS), f32)],
        scratch_types=[
            pltpu.VMEM((256, _DS), f32),
            pltpu.VMEM((2, 128), jnp.int32),
            pltpu.VMEM((2, 128), jnp.int32),
            pltpu.VMEM((128, _DS), f32),
            pltpu.VMEM((32, _DS), f32),
            pltpu.SemaphoreType.DMA,
            pltpu.VMEM_SHARED((_NC, _DS), f32),
        ],
    )
    _, cogg = run(pos16, idx_local, idx_global, zeros)
    return cogg


def _tc_body(pos_ref, cogg_ref, feat_ref, cidr_ref, cidc_ref, jlo_ref, jhi_ref,
             w1a_ref, w1an_ref, b1a_ref, g1a_ref, be1a_ref,
             w1br_ref, w1bf_ref, b1b_ref, g1b_ref, be1b_ref,
             w2a_ref, b2a_ref, g2a_ref, be2a_ref,
             w2br_ref, w2bf_ref, b2b_ref, g2b_ref, be2b_ref,
             wq_ref, wk_ref, wv_ref, wo_ref, bo_ref, gl_ref, bl_ref,
             out_ref,
             hpos_ref, q_ref, k_ref, v_ref, acc_ref, m_ref, l_ref):
    N = pos_ref.shape[1]
    NT = N // _T
    b = pl.program_id(0)
    f32 = jnp.float32

    bf16 = jnp.bfloat16

    def dot(a, bb, dims):
        return lax.dot_general(a, bb, (dims, ((), ())),
                               preferred_element_type=f32)

    def dotb(a, bb, dims):
        return dot(a.astype(bf16), bb, dims)

    def p1_tile(t):
        sl = pl.ds(t * _T, _T)
        pt = pos_ref[0, sl, :]
        cg = cogg_ref[0, sl, :]
        cnt = jnp.maximum(cg[:, 3:4], 1.0)
        lp = pt - cg / cnt
        n = jnp.sqrt(jnp.sum(lp * lp, axis=1, keepdims=True))
        x1 = dotb(lp, w1a_ref[...], ((1,), (0,))) + n * w1an_ref[...] + b1a_ref[...]
        r = jax.nn.relu(_ln(x1, g1a_ref[...], be1a_ref[...]))
        ft = feat_ref[0, sl, :]
        h1 = (dotb(r, w1br_ref[...], ((1,), (0,)))
              + dotb(ft, w1bf_ref[...], ((1,), (0,))) + b1b_ref[...])
        hpos = jax.nn.relu(_ln(h1, g1b_ref[...], be1b_ref[...]))
        x2 = dotb(lp, w2a_ref[...], ((1,), (0,))) + b2a_ref[...]
        rh = jax.nn.relu(_ln(x2, g2a_ref[...], be2a_ref[...]))
        h2 = (dotb(rh, w2br_ref[...], ((1,), (0,)))
              + dotb(ft, w2bf_ref[...], ((1,), (0,))) + b2b_ref[...])
        hgeo = jax.nn.relu(_ln(h2, g2b_ref[...], be2b_ref[...]))
        hpos_ref[sl, :] = hpos
        q_ref[sl, :] = (dotb(hgeo, wq_ref[...], ((1,), (0,))) * (1.0 / 16.0)).astype(bf16)
        k_ref[sl, :] = dotb(hgeo, wk_ref[...], ((1,), (0,))).astype(bf16)
        v_ref[sl, :] = dotb(hpos, wv_ref[...], ((1,), (0,))).astype(bf16)

    def p1(t, _):
        p1_tile(2 * t)
        p1_tile(2 * t + 1)
        return 0

    lax.fori_loop(0, NT // 2, p1, 0)

    def p2(t, _):
        sl = pl.ds(t * _T, _T)
        acc_ref[...] = jnp.zeros((_T, 256), f32)
        m_ref[...] = jnp.full((_T, 1), -1e20, f32)
        l_ref[...] = jnp.zeros((_T, 1), f32)
        cq = cidc_ref[0, sl, :]

        def jb(j, _):
            off = pl.multiple_of(j * _TK, _TK)
            ksl = pl.ds(off, _TK)
            s = dot(q_ref[sl, :], k_ref[ksl, :], ((1,), (1,)))
            ck = cidr_ref[0, :, ksl]
            s = jnp.where(cq == ck, s, -1e30)
            mprev = m_ref[...]
            mnew = jnp.maximum(mprev, jnp.max(s, axis=1, keepdims=True))
            p = jnp.exp(s - mnew)
            alpha = jnp.exp(mprev - mnew)
            l_ref[...] = l_ref[...] * alpha + jnp.sum(p, axis=1, keepdims=True)
            acc_ref[...] = acc_ref[...] * alpha + dotb(p, v_ref[ksl, :], ((1,), (0,)))
            m_ref[...] = mnew
            return 0

        lax.fori_loop(jlo_ref[b, t], jhi_ref[b, t] + 1, jb, 0)
        o = acc_ref[...] / l_ref[...]
        attn = dotb(o, wo_ref[...], ((1,), (0,))) + bo_ref[...]
        y = hpos_ref[sl, :] + attn
        out_ref[0, sl, :] = _ln(y, gl_ref[...], bl_ref[...])
        return 0

    lax.fori_loop(0, NT, p2, 0)


def kernel(pos, feat, cluster_ids, W1a, b1a, g1a, be1a, W1b, b1b, g1b, be1b,
           W2a, b2a, g2a, be2a, W2b, b2b, g2b, be2b, Wq, Wk, Wv, Wo, bo,
           g_ln1, b_ln1):
    B, N, _ = pos.shape
    D_PE = W1a.shape[1]
    D_FEAT = feat.shape[2]
    D_EMB = W1b.shape[1]
    NT = N // _T
    f32 = jnp.float32

    pos16 = jnp.zeros((B * N, _DS), f32).at[:, :3].set(pos.reshape(B * N, 3))
    pos16 = pos16.at[:, 3].set(1.0)
    cid = cluster_ids.astype(jnp.int32)
    idx_local = cid.reshape(B * N // 128, 128)
    idx_global = (cid + jnp.arange(B, dtype=jnp.int32)[:, None] * _NC
                  ).reshape(B * N // 128, 128)
    zeros = jnp.zeros((_NC, _DS), f32)

    cogg = _sc_segment(pos16, idx_local, idx_global, zeros).reshape(B, N, _DS)

    cid_row = cid.reshape(B, 1, N)
    cid_col = cid.reshape(B, N, 1)

    qs = jnp.arange(NT) * _T
    c_first = cid[:, qs]
    c_last = cid[:, qs + _T - 1]
    lo = jax.vmap(lambda a, v: jnp.searchsorted(a, v, side='left'))(cid, c_first)
    hi = jax.vmap(lambda a, v: jnp.searchsorted(a, v, side='right'))(cid, c_last)
    jlo = (lo // _TK).astype(jnp.int32)
    jhi = ((hi - 1) // _TK).astype(jnp.int32)

    W1a_pad = jnp.zeros((_DS, D_PE), f32).at[:3].set(W1a[:3])
    w1a_n = W1a[3:4]
    W2a_pad = jnp.zeros((_DS, D_PE), f32).at[:3].set(W2a[3:6])
    row = lambda x: x.reshape(1, -1)
    bf = lambda x: x.astype(jnp.bfloat16)

    grid = (B,)
    full = lambda shp: pl.BlockSpec(shp, lambda b: (0,) * len(shp))
    batched = lambda shp: pl.BlockSpec((1,) + shp, lambda b: (b, 0, 0))

    out = pl.pallas_call(
        _tc_body,
        grid=grid,
        in_specs=[
            batched((N, _DS)),
            batched((N, _DS)),
            batched((N, D_FEAT)),
            batched((1, N)),
            batched((N, 1)),
            pl.BlockSpec(memory_space=pltpu.SMEM),
            pl.BlockSpec(memory_space=pltpu.SMEM),
            full((_DS, D_PE)), full((1, D_PE)), full((1, D_PE)),
            full((1, D_PE)), full((1, D_PE)),
            full((D_PE, D_EMB)), full((D_FEAT, D_EMB)), full((1, D_EMB)),
            full((1, D_EMB)), full((1, D_EMB)),
            full((_DS, D_PE)), full((1, D_PE)), full((1, D_PE)), full((1, D_PE)),
            full((D_PE, D_EMB)), full((D_FEAT, D_EMB)), full((1, D_EMB)),
            full((1, D_EMB)), full((1, D_EMB)),
            full((D_EMB, D_EMB)), full((D_EMB, D_EMB)), full((D_EMB, D_EMB)),
            full((D_EMB, D_EMB)), full((1, D_EMB)), full((1, D_EMB)),
            full((1, D_EMB)),
        ],
        out_specs=batched((N, D_EMB)),
        out_shape=jax.ShapeDtypeStruct((B, N, D_EMB), f32),
        scratch_shapes=[
            pltpu.VMEM((N, D_EMB), f32),
            pltpu.VMEM((N, D_EMB), jnp.bfloat16),
            pltpu.VMEM((N, D_EMB), jnp.bfloat16),
            pltpu.VMEM((N, D_EMB), jnp.bfloat16),
            pltpu.VMEM((_T, D_EMB), f32),
            pltpu.VMEM((_T, 1), f32),
            pltpu.VMEM((_T, 1), f32),
        ],
    )(pos16.reshape(B, N, _DS), cogg, feat, cid_row, cid_col, jlo, jhi,
      bf(W1a_pad), w1a_n, row(b1a), row(g1a), row(be1a),
      bf(W1b[:D_PE]), bf(W1b[D_PE:]), row(b1b), row(g1b), row(be1b),
      bf(W2a_pad), row(b2a), row(g2a), row(be2a),
      bf(W2b[:D_PE]), bf(W2b[D_PE:]), row(b2b), row(g2b), row(be2b),
      bf(Wq), bf(Wk), bf(Wv), bf(Wo), row(bo), row(g_ln1), row(b_ln1))
    return out


# speedup vs baseline: 1.2354x; 1.1647x over previous
import functools

import jax
import jax.numpy as jnp
from jax import lax
from jax.experimental import pallas as pl
from jax.experimental.pallas import tpu as pltpu
from jax.experimental.pallas import tpu_sc as plsc

_NC = 512
_T = 256
_TK = 256
_DS = 128


def _ln(x, g, b):
    m = jnp.mean(x, axis=-1, keepdims=True)
    v = jnp.mean((x - m) * (x - m), axis=-1, keepdims=True)
    return (x - m) * jax.lax.rsqrt(v + 1e-5) * g + b


def _sc_body(pos_hbm, idxl_hbm, idxg_hbm, zeros_hbm, sums_hbm, cogg_hbm,
             pos_v, idxl_v, idxg_v, rows_v, acc32_v, sem, acc_sh):
    c = lax.axis_index("c")
    s = lax.axis_index("s")

    @pl.when(s == 0)
    def _():
        pltpu.sync_copy(zeros_hbm, acc_sh)
    plsc.subcore_barrier()

    base = c * 32 + s * 2
    pltpu.sync_copy(pos_hbm.at[pl.ds(base * 128, 256)], pos_v)
    pltpu.sync_copy(idxl_hbm.at[pl.ds(base, 2)], idxl_v)
    pltpu.sync_copy(idxg_hbm.at[pl.ds(base, 2)], idxg_v)
    for j in range(2):
        pltpu.sync_copy(pos_v.at[pl.ds(j * 128, 128)],
                        acc_sh.at[idxl_v.at[j]], add=True)
    plsc.subcore_barrier()

    pltpu.sync_copy(acc_sh.at[pl.ds(s * 32, 32)], acc32_v)
    pltpu.sync_copy(acc32_v, sums_hbm.at[pl.ds(c * _NC + s * 32, 32)])
    plsc.subcore_barrier()
    for j in range(2):
        pltpu.async_copy(sums_hbm.at[idxg_v.at[j]], rows_v, sem).wait()
        pltpu.sync_copy(rows_v, cogg_hbm.at[pl.ds((base + j) * 128, 128)])


def _sc_segment(pos16, idx_local, idx_global, zeros):
    BN = pos16.shape[0]
    mesh = plsc.VectorSubcoreMesh(core_axis_name="c", subcore_axis_name="s")
    f32 = jnp.float32
    run = pl.kernel(
        _sc_body,
        mesh=mesh,
        out_type=[jax.ShapeDtypeStruct((2 * _NC, _DS), f32),
                  jax.ShapeDtypeStruct((BN, _DS), f32)],
        scratch_types=[
            pltpu.VMEM((256, _DS), f32),
            pltpu.VMEM((2, 128), jnp.int32),
            pltpu.VMEM((2, 128), jnp.int32),
            pltpu.VMEM((128, _DS), f32),
            pltpu.VMEM((32, _DS), f32),
            pltpu.SemaphoreType.DMA,
            pltpu.VMEM_SHARED((_NC, _DS), f32),
        ],
    )
    _, cogg = run(pos16, idx_local, idx_global, zeros)
    return cogg


def _tc_body(pos_ref, cogg_ref, feat_ref, cidr_ref, cidc_ref, jlo_ref, jhi_ref,
             w1a_ref, w1an_ref, b1a_ref, g1a_ref, be1a_ref,
             w1br_ref, w1bf_ref, b1b_ref, g1b_ref, be1b_ref,
             w2a_ref, b2a_ref, g2a_ref, be2a_ref,
             w2br_ref, w2bf_ref, b2b_ref, g2b_ref, be2b_ref,
             wq_ref, wk_ref, wv_ref, wo_ref, bo_ref, gl_ref, bl_ref,
             out_ref,
             hpos_ref, q_ref, k_ref, v_ref, acc_ref, m_ref, l_ref):
    N = pos_ref.shape[1]
    NT = N // _T
    b = pl.program_id(0)
    f32 = jnp.float32

    def dot(a, bb, dims):
        return lax.dot_general(a, bb, (dims, ((), ())),
                               preferred_element_type=f32)

    def p1_tile(t):
        sl = pl.ds(t * _T, _T)
        pt = pos_ref[0, sl, :]
        cg = cogg_ref[0, sl, :]
        cnt = jnp.maximum(cg[:, 3:4], 1.0)
        lp = pt - cg / cnt
        n = jnp.sqrt(jnp.sum(lp * lp, axis=1, keepdims=True))
        x1 = dot(lp, w1a_ref[...], ((1,), (0,))) + n * w1an_ref[...] + b1a_ref[...]
        r = jax.nn.relu(_ln(x1, g1a_ref[...], be1a_ref[...]))
        ft = feat_ref[0, sl, :]
        h1 = (dot(r, w1br_ref[...], ((1,), (0,)))
              + dot(ft, w1bf_ref[...], ((1,), (0,))) + b1b_ref[...])
        hpos = jax.nn.relu(_ln(h1, g1b_ref[...], be1b_ref[...]))
        x2 = dot(lp, w2a_ref[...], ((1,), (0,))) + b2a_ref[...]
        rh = jax.nn.relu(_ln(x2, g2a_ref[...], be2a_ref[...]))
        h2 = (dot(rh, w2br_ref[...], ((1,), (0,)))
              + dot(ft, w2bf_ref[...], ((1,), (0,))) + b2b_ref[...])
        hgeo = jax.nn.relu(_ln(h2, g2b_ref[...], be2b_ref[...]))
        hpos_ref[sl, :] = hpos
        q_ref[sl, :] = dot(hgeo, wq_ref[...], ((1,), (0,))) * (1.0 / 16.0)
        k_ref[sl, :] = dot(hgeo, wk_ref[...], ((1,), (0,)))
        v_ref[sl, :] = dot(hpos, wv_ref[...], ((1,), (0,)))

    def p1(t, _):
        p1_tile(2 * t)
        p1_tile(2 * t + 1)
        return 0

    lax.fori_loop(0, NT // 2, p1, 0)

    NTK = N // _TK
    def p2(t, _):
        sl = pl.ds(t * _T, _T)
        cq = cidc_ref[0, sl, :]
        qt = q_ref[sl, :]
        j0 = jnp.minimum(jlo_ref[b, t], NTK - 2)
        off0 = pl.multiple_of(j0 * _TK, _TK)
        ksl0 = pl.ds(off0, 2 * _TK)
        s0 = dot(qt, k_ref[ksl0, :], ((1,), (1,)))
        ck0 = cidr_ref[0, :, ksl0]
        s0 = jnp.where(cq == ck0, s0, -1e30)
        m0 = jnp.maximum(jnp.max(s0, axis=1, keepdims=True), -1e20)
        p0 = jnp.exp(s0 - m0)
        m_ref[...] = m0
        l_ref[...] = jnp.sum(p0, axis=1, keepdims=True)
        acc_ref[...] = dot(p0, v_ref[ksl0, :], ((1,), (0,)))

        def jb(j, _):
            off = pl.multiple_of(j * _TK, _TK)
            ksl = pl.ds(off, _TK)
            s = dot(qt, k_ref[ksl, :], ((1,), (1,)))
            ck = cidr_ref[0, :, ksl]
            s = jnp.where(cq == ck, s, -1e30)
            mprev = m_ref[...]
            mnew = jnp.maximum(mprev, jnp.max(s, axis=1, keepdims=True))
            p = jnp.exp(s - mnew)
            alpha = jnp.exp(mprev - mnew)
            l_ref[...] = l_ref[...] * alpha + jnp.sum(p, axis=1, keepdims=True)
            acc_ref[...] = acc_ref[...] * alpha + dot(p, v_ref[ksl, :], ((1,), (0,)))
            m_ref[...] = mnew
            return 0

        lax.fori_loop(jlo_ref[b, t] + 2, jhi_ref[b, t] + 1, jb, 0)
        o = acc_ref[...] / l_ref[...]
        attn = dot(o, wo_ref[...], ((1,), (0,))) + bo_ref[...]
        y = hpos_ref[sl, :] + attn
        out_ref[0, sl, :] = _ln(y, gl_ref[...], bl_ref[...])
        return 0

    lax.fori_loop(0, NT, p2, 0)


def kernel(pos, feat, cluster_ids, W1a, b1a, g1a, be1a, W1b, b1b, g1b, be1b,
           W2a, b2a, g2a, be2a, W2b, b2b, g2b, be2b, Wq, Wk, Wv, Wo, bo,
           g_ln1, b_ln1):
    B, N, _ = pos.shape
    D_PE = W1a.shape[1]
    D_FEAT = feat.shape[2]
    D_EMB = W1b.shape[1]
    NT = N // _T
    f32 = jnp.float32

    pos16 = jnp.zeros((B * N, _DS), f32).at[:, :3].set(pos.reshape(B * N, 3))
    pos16 = pos16.at[:, 3].set(1.0)
    cid = cluster_ids.astype(jnp.int32)
    idx_local = cid.reshape(B * N // 128, 128)
    idx_global = (cid + jnp.arange(B, dtype=jnp.int32)[:, None] * _NC
                  ).reshape(B * N // 128, 128)
    zeros = jnp.zeros((_NC, _DS), f32)

    cogg = _sc_segment(pos16, idx_local, idx_global, zeros).reshape(B, N, _DS)

    cid_row = cid.reshape(B, 1, N)
    cid_col = cid.reshape(B, N, 1)

    qs = jnp.arange(NT) * _T
    c_first = cid[:, qs]
    c_last = cid[:, qs + _T - 1]
    lo = jax.vmap(lambda a, v: jnp.searchsorted(a, v, side='left'))(cid, c_first)
    hi = jax.vmap(lambda a, v: jnp.searchsorted(a, v, side='right'))(cid, c_last)
    jlo = (lo // _TK).astype(jnp.int32)
    jhi = ((hi - 1) // _TK).astype(jnp.int32)

    W1a_pad = jnp.zeros((_DS, D_PE), f32).at[:3].set(W1a[:3])
    w1a_n = W1a[3:4]
    W2a_pad = jnp.zeros((_DS, D_PE), f32).at[:3].set(W2a[3:6])
    row = lambda x: x.reshape(1, -1)

    grid = (B,)
    full = lambda shp: pl.BlockSpec(shp, lambda b: (0,) * len(shp))
    batched = lambda shp: pl.BlockSpec((1,) + shp, lambda b: (b, 0, 0))

    out = pl.pallas_call(
        _tc_body,
        grid=grid,
        in_specs=[
            batched((N, _DS)),
            batched((N, _DS)),
            batched((N, D_FEAT)),
            batched((1, N)),
            batched((N, 1)),
            pl.BlockSpec(memory_space=pltpu.SMEM),
            pl.BlockSpec(memory_space=pltpu.SMEM),
            full((_DS, D_PE)), full((1, D_PE)), full((1, D_PE)),
            full((1, D_PE)), full((1, D_PE)),
            full((D_PE, D_EMB)), full((D_FEAT, D_EMB)), full((1, D_EMB)),
            full((1, D_EMB)), full((1, D_EMB)),
            full((_DS, D_PE)), full((1, D_PE)), full((1, D_PE)), full((1, D_PE)),
            full((D_PE, D_EMB)), full((D_FEAT, D_EMB)), full((1, D_EMB)),
            full((1, D_EMB)), full((1, D_EMB)),
            full((D_EMB, D_EMB)), full((D_EMB, D_EMB)), full((D_EMB, D_EMB)),
            full((D_EMB, D_EMB)), full((1, D_EMB)), full((1, D_EMB)),
            full((1, D_EMB)),
        ],
        out_specs=batched((N, D_EMB)),
        out_shape=jax.ShapeDtypeStruct((B, N, D_EMB), f32),
        scratch_shapes=[
            pltpu.VMEM((N, D_EMB), f32),
            pltpu.VMEM((N, D_EMB), f32),
            pltpu.VMEM((N, D_EMB), f32),
            pltpu.VMEM((N, D_EMB), f32),
            pltpu.VMEM((_T, D_EMB), f32),
            pltpu.VMEM((_T, 1), f32),
            pltpu.VMEM((_T, 1), f32),
        ],
    )(pos16.reshape(B, N, _DS), cogg, feat, cid_row, cid_col, jlo, jhi,
      W1a_pad, w1a_n, row(b1a), row(g1a), row(be1a),
      W1b[:D_PE], W1b[D_PE:], row(b1b), row(g1b), row(be1b),
      W2a_pad, row(b2a), row(g2a), row(be2a),
      W2b[:D_PE], W2b[D_PE:], row(b2b), row(g2b), row(be2b),
      Wq, Wk, Wv, Wo, row(bo), row(g_ln1), row(b_ln1))
    return out
